# Initial kernel scaffold; baseline (speedup 1.0000x reference)
import jax
import jax.numpy as jnp
from jax.experimental import pallas as pl


def kernel(card_ids, mana_costs, card_types, powers, toughnesses, card_table, mana_table, type_table, power_table, tough_table, W, b):
    raise NotImplementedError("write your pallas kernel here")


# speedup vs baseline: 4.7822x; 4.7822x over previous
import functools

import jax
import jax.numpy as jnp
from jax import lax
from jax.experimental import pallas as pl
from jax.experimental.pallas import tpu as pltpu
from jax.experimental.pallas import tpu_sc as plsc

_NC = 2
_NS = 16
_NW = _NC * _NS
_SUB = 128
_G = 8


def _sc_gather(idx2d, table):
    n_sub = idx2d.shape[0]
    d = table.shape[1]
    per_w = n_sub // _NW
    n_grp = per_w // _G
    mesh = plsc.VectorSubcoreMesh(core_axis_name="c", subcore_axis_name="s")

    @functools.partial(
        pl.kernel,
        out_type=jax.ShapeDtypeStruct((n_sub * _SUB, d), table.dtype),
        mesh=mesh,
        scratch_types=[
            pltpu.VMEM((per_w, _SUB), jnp.int32),
            pltpu.VMEM((_G * _SUB, d), table.dtype),
            pltpu.SemaphoreType.DMA,
        ],
        compiler_params=pltpu.CompilerParams(use_tc_tiling_on_sc=False),
    )
    def k(idx_hbm, table_hbm, out_hbm, idx_v, rows_v, sem):
        wid = lax.axis_index("s") * _NC + lax.axis_index("c")
        pltpu.sync_copy(idx_hbm.at[pl.ds(wid * per_w, per_w)], idx_v)

        def body(g, carry):
            copies = []
            for j in range(_G):
                copies.append(pltpu.async_copy(
                    table_hbm.at[idx_v.at[g * _G + j]],
                    rows_v.at[pl.ds(j * _SUB, _SUB)], sem))
            for c in copies:
                c.wait()
            base = (wid * per_w + g * _G) * _SUB
            pltpu.sync_copy(rows_v, out_hbm.at[pl.ds(base, _G * _SUB)])
            return carry

        lax.fori_loop(0, n_grp, body, 0)

    return k(idx2d, table)


def _tc_body(n_rows, card_ref, m_ref, t_ref, p_ref, tg_ref, s_ref, w_ref,
             b_ref, o_ref, *, offs):
    om, ot, op_, otg = offs
    iota = lax.broadcasted_iota(jnp.int32, (card_ref.shape[0], s_ref.shape[0]), 1)
    oh = ((iota == m_ref[...] + om) | (iota == t_ref[...] + ot)
          | (iota == p_ref[...] + op_) | (iota == tg_ref[...] + otg))
    small = jnp.dot(oh.astype(jnp.float32), s_ref[...],
                    preferred_element_type=jnp.float32)
    comb = jnp.concatenate([card_ref[...], small], axis=1)
    o_ref[...] = jnp.dot(comb, w_ref[...],
                         preferred_element_type=jnp.float32) + b_ref[...]


def _tc_combine(card_emb, m_idx, t_idx, p_idx, tg_idx, s_tab, w, b, offs,
                interpret=False):
    n, dc = card_emb.shape
    bs = 512
    grid = n // bs
    blk = lambda i: (i, 0)
    full = lambda i: (0, 0)
    body = functools.partial(_tc_body, bs, offs=offs)
    return pl.pallas_call(
        body,
        grid=(grid,),
        in_specs=[
            pl.BlockSpec((bs, dc), blk),
            pl.BlockSpec((bs, 1), blk),
            pl.BlockSpec((bs, 1), blk),
            pl.BlockSpec((bs, 1), blk),
            pl.BlockSpec((bs, 1), blk),
            pl.BlockSpec(s_tab.shape, full),
            pl.BlockSpec(w.shape, full),
            pl.BlockSpec((1, w.shape[1]), full),
        ],
        out_specs=pl.BlockSpec((bs, w.shape[1]), blk),
        out_shape=jax.ShapeDtypeStruct((n, w.shape[1]), jnp.float32),
        interpret=interpret,
    )(card_emb, m_idx, t_idx, p_idx, tg_idx, s_tab, w, b)


def kernel(card_ids, mana_costs, card_types, powers, toughnesses,
           card_table, mana_table, type_table, power_table, tough_table, W, b):
    bsz, seq = card_ids.shape
    n = bsz * seq
    d_card = card_table.shape[1]
    d_mana, d_type, d_pt = mana_table.shape[1], type_table.shape[1], power_table.shape[1]

    om = 0
    ot = om + mana_table.shape[0]
    op_ = ot + type_table.shape[0]
    otg = op_ + power_table.shape[0]
    rows = otg + tough_table.shape[0]
    rows_pad = ((rows + 63) // 64) * 64
    d_small = d_mana + d_type + 2 * d_pt
    s_tab = jnp.zeros((rows_pad, d_small), jnp.float32)
    s_tab = (s_tab.at[om:om + mana_table.shape[0], 0:d_mana].set(mana_table)
             .at[ot:ot + type_table.shape[0], d_mana:d_mana + d_type].set(type_table)
             .at[op_:op_ + power_table.shape[0],
                 d_mana + d_type:d_mana + d_type + d_pt].set(power_table)
             .at[otg:otg + tough_table.shape[0],
                 d_mana + d_type + d_pt:d_small].set(tough_table))

    idx2d = card_ids.reshape(n // _SUB, _SUB).astype(jnp.int32)
    card_emb = _sc_gather(idx2d, card_table)

    col = lambda a: a.reshape(n, 1).astype(jnp.int32)
    out = _tc_combine(card_emb, col(mana_costs), col(card_types), col(powers),
                      col(toughnesses), s_tab, W, b.reshape(1, -1),
                      (om, ot, op_, otg))
    return out.reshape(bsz, seq, W.shape[1])


# speedup vs baseline: 6.2793x; 1.3131x over previous
import functools

import jax
import jax.numpy as jnp
from jax import lax
from jax.experimental import pallas as pl
from jax.experimental.pallas import tpu as pltpu
from jax.experimental.pallas import tpu_sc as plsc

_NC = 2
_NS = 16
_NW = _NC * _NS
_SUB = 128
_G = 8


def _sc_gather(idx2d, table):
    n_sub = idx2d.shape[0]
    d = table.shape[1]
    per_w = n_sub // _NW
    n_grp = per_w // _G
    mesh = plsc.VectorSubcoreMesh(core_axis_name="c", subcore_axis_name="s")

    @functools.partial(
        pl.kernel,
        out_type=jax.ShapeDtypeStruct((n_sub * _SUB, d), table.dtype),
        mesh=mesh,
        scratch_types=[
            pltpu.VMEM((per_w, _SUB), jnp.int32),
            pltpu.VMEM((_G * _SUB, d), table.dtype),
            pltpu.SemaphoreType.DMA,
        ],
        compiler_params=pltpu.CompilerParams(use_tc_tiling_on_sc=False),
    )
    def k(idx_hbm, table_hbm, out_hbm, idx_v, rows_v, sem):
        wid = lax.axis_index("s") * _NC + lax.axis_index("c")
        pltpu.sync_copy(idx_hbm.at[pl.ds(wid * per_w, per_w)], idx_v)

        def body(g, carry):
            copies = []
            for j in range(_G):
                copies.append(pltpu.async_copy(
                    table_hbm.at[idx_v.at[g * _G + j]],
                    rows_v.at[pl.ds(j * _SUB, _SUB)], sem))
            for c in copies:
                c.wait()
            base = (wid * per_w + g * _G) * _SUB
            pltpu.sync_copy(rows_v, out_hbm.at[pl.ds(base, _G * _SUB)])
            return carry

        lax.fori_loop(0, n_grp, body, 0)

    return k(idx2d, table)


def _tc_body(n_rows, card_ref, m_ref, t_ref, p_ref, tg_ref, s_ref, w_ref,
             b_ref, o_ref, *, offs):
    om, ot, op_, otg = offs
    iota = lax.broadcasted_iota(jnp.int32, (card_ref.shape[0], s_ref.shape[0]), 1)
    oh = ((iota == m_ref[...] + om) | (iota == t_ref[...] + ot)
          | (iota == p_ref[...] + op_) | (iota == tg_ref[...] + otg))
    small = jnp.dot(oh.astype(jnp.float32), s_ref[...],
                    preferred_element_type=jnp.float32)
    comb = jnp.concatenate([card_ref[...], small], axis=1)
    o_ref[...] = jnp.dot(comb, w_ref[...],
                         preferred_element_type=jnp.float32) + b_ref[...]


def _tc_combine(card_emb, m_idx, t_idx, p_idx, tg_idx, s_tab, w, b, offs,
                interpret=False):
    n, dc = card_emb.shape
    bs = 2048
    grid = n // bs
    blk = lambda i: (i, 0)
    full = lambda i: (0, 0)
    body = functools.partial(_tc_body, bs, offs=offs)
    return pl.pallas_call(
        body,
        grid=(grid,),
        in_specs=[
            pl.BlockSpec((bs, dc), blk),
            pl.BlockSpec((bs, 1), blk),
            pl.BlockSpec((bs, 1), blk),
            pl.BlockSpec((bs, 1), blk),
            pl.BlockSpec((bs, 1), blk),
            pl.BlockSpec(s_tab.shape, full),
            pl.BlockSpec(w.shape, full),
            pl.BlockSpec((1, w.shape[1]), full),
        ],
        out_specs=pl.BlockSpec((bs, w.shape[1]), blk),
        out_shape=jax.ShapeDtypeStruct((n, w.shape[1]), jnp.float32),
        interpret=interpret,
    )(card_emb, m_idx, t_idx, p_idx, tg_idx, s_tab, w, b)


def kernel(card_ids, mana_costs, card_types, powers, toughnesses,
           card_table, mana_table, type_table, power_table, tough_table, W, b):
    bsz, seq = card_ids.shape
    n = bsz * seq
    d_card = card_table.shape[1]
    d_mana, d_type, d_pt = mana_table.shape[1], type_table.shape[1], power_table.shape[1]

    om = 0
    ot = om + mana_table.shape[0]
    op_ = ot + type_table.shape[0]
    otg = op_ + power_table.shape[0]
    rows = otg + tough_table.shape[0]
    rows_pad = ((rows + 63) // 64) * 64
    d_small = d_mana + d_type + 2 * d_pt
    s_tab = jnp.zeros((rows_pad, d_small), jnp.float32)
    s_tab = (s_tab.at[om:om + mana_table.shape[0], 0:d_mana].set(mana_table)
             .at[ot:ot + type_table.shape[0], d_mana:d_mana + d_type].set(type_table)
             .at[op_:op_ + power_table.shape[0],
                 d_mana + d_type:d_mana + d_type + d_pt].set(power_table)
             .at[otg:otg + tough_table.shape[0],
                 d_mana + d_type + d_pt:d_small].set(tough_table))

    idx2d = card_ids.reshape(n // _SUB, _SUB).astype(jnp.int32)
    card_emb = _sc_gather(idx2d, card_table)

    col = lambda a: a.reshape(n, 1).astype(jnp.int32)
    out = _tc_combine(card_emb, col(mana_costs), col(card_types), col(powers),
                      col(toughnesses), s_tab, W, b.reshape(1, -1),
                      (om, ot, op_, otg))
    return out.reshape(bsz, seq, W.shape[1])


# speedup vs baseline: 7.5046x; 1.1951x over previous
import functools

import jax
import jax.numpy as jnp
from jax import lax
from jax.experimental import pallas as pl
from jax.experimental.pallas import tpu as pltpu
from jax.experimental.pallas import tpu_sc as plsc

_NC = 2
_NS = 16
_NW = _NC * _NS
_SUB = 128
_G = 8


def _sc_gather3(idx0, idx1, idx2, tab0, tab1, tab2):
    n_sub = idx0.shape[0]
    d0, d1, d2 = tab0.shape[1], tab1.shape[1], tab2.shape[1]
    per_w = n_sub // _NW
    n_grp = per_w // _G
    mesh = plsc.VectorSubcoreMesh(core_axis_name="c", subcore_axis_name="s")

    @functools.partial(
        pl.kernel,
        out_type=(
            jax.ShapeDtypeStruct((n_sub * _SUB, d0), tab0.dtype),
            jax.ShapeDtypeStruct((n_sub * _SUB, d1), tab1.dtype),
            jax.ShapeDtypeStruct((n_sub * _SUB, d2), tab2.dtype),
        ),
        mesh=mesh,
        scratch_types=[
            pltpu.VMEM((_G, _SUB), jnp.int32),
            pltpu.VMEM((_G, _SUB), jnp.int32),
            pltpu.VMEM((_G, _SUB), jnp.int32),
            pltpu.VMEM((_G * _SUB, d0), tab0.dtype),
            pltpu.VMEM((_G * _SUB, d1), tab1.dtype),
            pltpu.VMEM((_G * _SUB, d2), tab2.dtype),
            pltpu.SemaphoreType.DMA,
            pltpu.SemaphoreType.DMA,
        ],
        compiler_params=pltpu.CompilerParams(use_tc_tiling_on_sc=False),
    )
    def k(i0_hbm, i1_hbm, i2_hbm, t0_hbm, t1_hbm, t2_hbm,
          o0_hbm, o1_hbm, o2_hbm, i0v, i1v, i2v, r0v, r1v, r2v, isem, gsem):
        wid = lax.axis_index("s") * _NC + lax.axis_index("c")

        def body(g, carry):
            base_sub = wid * per_w + g * _G
            ic = [pltpu.async_copy(i0_hbm.at[pl.ds(base_sub, _G)], i0v, isem),
                  pltpu.async_copy(i1_hbm.at[pl.ds(base_sub, _G)], i1v, isem),
                  pltpu.async_copy(i2_hbm.at[pl.ds(base_sub, _G)], i2v, isem)]
            for c in ic:
                c.wait()
            copies = []
            for j in range(_G):
                sl = pl.ds(j * _SUB, _SUB)
                copies.append(pltpu.async_copy(t0_hbm.at[i0v.at[j]], r0v.at[sl], gsem))
                copies.append(pltpu.async_copy(t1_hbm.at[i1v.at[j]], r1v.at[sl], gsem))
                copies.append(pltpu.async_copy(t2_hbm.at[i2v.at[j]], r2v.at[sl], gsem))
            for c in copies:
                c.wait()
            base = base_sub * _SUB
            sl = pl.ds(base, _G * _SUB)
            pltpu.sync_copy(r0v, o0_hbm.at[sl])
            pltpu.sync_copy(r1v, o1_hbm.at[sl])
            pltpu.sync_copy(r2v, o2_hbm.at[sl])
            return carry

        lax.fori_loop(0, n_grp, body, 0)

    return k(idx0, idx1, idx2, tab0, tab1, tab2)


def _tc_body(card_ref, mt_ref, pt_ref, w0_ref, w1_ref, w2_ref, b_ref, o_ref):
    acc = jnp.dot(card_ref[...], w0_ref[...], preferred_element_type=jnp.float32)
    acc += jnp.dot(mt_ref[...], w1_ref[...], preferred_element_type=jnp.float32)
    acc += jnp.dot(pt_ref[...], w2_ref[...], preferred_element_type=jnp.float32)
    o_ref[...] = acc + b_ref[...]


def _tc_combine(card_emb, mt_emb, pt_emb, w0, w1, w2, b, interpret=False):
    n = card_emb.shape[0]
    bs = 2048
    blk = lambda i: (i, 0)
    full = lambda i: (0, 0)
    d_out = w0.shape[1]
    return pl.pallas_call(
        _tc_body,
        grid=(n // bs,),
        in_specs=[
            pl.BlockSpec((bs, card_emb.shape[1]), blk),
            pl.BlockSpec((bs, mt_emb.shape[1]), blk),
            pl.BlockSpec((bs, pt_emb.shape[1]), blk),
            pl.BlockSpec(w0.shape, full),
            pl.BlockSpec(w1.shape, full),
            pl.BlockSpec(w2.shape, full),
            pl.BlockSpec((1, d_out), full),
        ],
        out_specs=pl.BlockSpec((bs, d_out), blk),
        out_shape=jax.ShapeDtypeStruct((n, d_out), jnp.float32),
        interpret=interpret,
    )(card_emb, mt_emb, pt_emb, w0, w1, w2, b)


def kernel(card_ids, mana_costs, card_types, powers, toughnesses,
           card_table, mana_table, type_table, power_table, tough_table, W, b):
    bsz, seq = card_ids.shape
    n = bsz * seq
    d_card = card_table.shape[1]
    n_mana, d_mana = mana_table.shape
    n_type, d_type = type_table.shape
    n_pow, d_pt = power_table.shape
    n_tgh = tough_table.shape[0]

    mt_tab = jnp.concatenate(
        [jnp.repeat(mana_table, n_type, axis=0), jnp.tile(type_table, (n_mana, 1))],
        axis=1)
    pt_tab = jnp.concatenate(
        [jnp.repeat(power_table, n_tgh, axis=0), jnp.tile(tough_table, (n_pow, 1))],
        axis=1)

    to2d = lambda a: a.reshape(n // _SUB, _SUB).astype(jnp.int32)
    card_idx = to2d(card_ids)
    mt_idx = to2d(mana_costs * n_type + card_types)
    pt_idx = to2d(powers * n_tgh + toughnesses)

    card_emb, mt_emb, pt_emb = _sc_gather3(
        card_idx, mt_idx, pt_idx, card_table, mt_tab, pt_tab)

    d_mt = d_mana + d_type
    w0 = W[:d_card]
    w1 = W[d_card:d_card + d_mt]
    w2 = W[d_card + d_mt:]
    out = _tc_combine(card_emb, mt_emb, pt_emb, w0, w1, w2, b.reshape(1, -1))
    return out.reshape(bsz, seq, W.shape[1])


# speedup vs baseline: 14.4289x; 1.9227x over previous
import functools

import jax
import jax.numpy as jnp
from jax import lax
from jax.experimental import pallas as pl
from jax.experimental.pallas import tpu as pltpu
from jax.experimental.pallas import tpu_sc as plsc

_NC = 2
_NS = 16
_NW = _NC * _NS
_SUB = 128
_G = 5


def _sc_gather3(idx0, idx1, idx2, tab0, tab1, tab2):
    n_sub = idx0.shape[0]
    d0, d1, d2 = tab0.shape[1], tab1.shape[1], tab2.shape[1]
    per_w = n_sub // _NW
    n_grp = per_w // _G
    mesh = plsc.VectorSubcoreMesh(core_axis_name="c", subcore_axis_name="s")

    @functools.partial(
        pl.kernel,
        out_type=jax.ShapeDtypeStruct((n_sub * _SUB, d0 + d1 + d2), tab0.dtype),
        mesh=mesh,
        scratch_types=[
            pltpu.VMEM((_G, _SUB), jnp.int32),
            pltpu.VMEM((_G, _SUB), jnp.int32),
            pltpu.VMEM((_G, _SUB), jnp.int32),
            pltpu.VMEM((_G * _SUB, d0), tab0.dtype),
            pltpu.VMEM((_G * _SUB, d1), tab1.dtype),
            pltpu.VMEM((_G * _SUB, d2), tab2.dtype),
            pltpu.SemaphoreType.DMA,
            pltpu.SemaphoreType.DMA,
        ],
        compiler_params=pltpu.CompilerParams(use_tc_tiling_on_sc=False),
    )
    def k(i0_hbm, i1_hbm, i2_hbm, t0_hbm, t1_hbm, t2_hbm,
          o_hbm, i0v, i1v, i2v, r0v, r1v, r2v, isem, gsem):
        wid = lax.axis_index("s") * _NC + lax.axis_index("c")

        def body(g, carry):
            base_sub = wid * per_w + g * _G
            ic = [pltpu.async_copy(i0_hbm.at[pl.ds(base_sub, _G)], i0v, isem),
                  pltpu.async_copy(i1_hbm.at[pl.ds(base_sub, _G)], i1v, isem),
                  pltpu.async_copy(i2_hbm.at[pl.ds(base_sub, _G)], i2v, isem)]
            for c in ic:
                c.wait()
            copies = []
            for j in range(_G):
                sl = pl.ds(j * _SUB, _SUB)
                copies.append(pltpu.async_copy(t0_hbm.at[i0v.at[j]], r0v.at[sl], gsem))
                copies.append(pltpu.async_copy(t1_hbm.at[i1v.at[j]], r1v.at[sl], gsem))
                copies.append(pltpu.async_copy(t2_hbm.at[i2v.at[j]], r2v.at[sl], gsem))
            for c in copies:
                c.wait()
            rows = pl.ds(base_sub * _SUB, _G * _SUB)
            pltpu.sync_copy(r0v, o_hbm.at[rows, pl.ds(0, d0)])
            pltpu.sync_copy(r1v, o_hbm.at[rows, pl.ds(d0, d1)])
            pltpu.sync_copy(r2v, o_hbm.at[rows, pl.ds(d0 + d1, d2)])
            return carry

        lax.fori_loop(0, n_grp, body, 0)

    return k(idx0, idx1, idx2, tab0, tab1, tab2)


def _tc_body(comb_ref, w_ref, b_ref, o_ref):
    o_ref[...] = jnp.dot(comb_ref[...], w_ref[...],
                         preferred_element_type=jnp.float32) + b_ref[...]


def _tc_combine(comb, w_pad, b, interpret=False):
    n = comb.shape[0]
    bs = 2048
    blk = lambda i: (i, 0)
    full = lambda i: (0, 0)
    d_out = w_pad.shape[1]
    return pl.pallas_call(
        _tc_body,
        grid=(n // bs,),
        in_specs=[
            pl.BlockSpec((bs, comb.shape[1]), blk),
            pl.BlockSpec(w_pad.shape, full),
            pl.BlockSpec((1, d_out), full),
        ],
        out_specs=pl.BlockSpec((bs, d_out), blk),
        out_shape=jax.ShapeDtypeStruct((n, d_out), jnp.float32),
        interpret=interpret,
    )(comb, w_pad, b)


def kernel(card_ids, mana_costs, card_types, powers, toughnesses,
           card_table, mana_table, type_table, power_table, tough_table, W, b):
    bsz, seq = card_ids.shape
    n = bsz * seq
    d_card = card_table.shape[1]
    n_mana, d_mana = mana_table.shape
    n_type, d_type = type_table.shape
    n_pow, d_pt = power_table.shape
    n_tgh = tough_table.shape[0]

    mt_tab = jnp.concatenate(
        [jnp.repeat(mana_table, n_type, axis=0), jnp.tile(type_table, (n_mana, 1))],
        axis=1)
    d_mt = d_mana + d_type
    d_ptp = 128 - d_card - d_mt
    pt_tab = jnp.concatenate(
        [jnp.repeat(power_table, n_tgh, axis=0), jnp.tile(tough_table, (n_pow, 1)),
         jnp.zeros((n_pow * n_tgh, d_ptp - 2 * d_pt), power_table.dtype)],
        axis=1)

    to2d = lambda a: a.reshape(n // _SUB, _SUB).astype(jnp.int32)
    card_idx = to2d(card_ids)
    mt_idx = to2d(mana_costs * n_type + card_types)
    pt_idx = to2d(powers * n_tgh + toughnesses)

    comb = _sc_gather3(card_idx, mt_idx, pt_idx, card_table, mt_tab, pt_tab)

    w_pad = jnp.concatenate(
        [W, jnp.zeros((128 - W.shape[0], W.shape[1]), W.dtype)], axis=0)
    out = _tc_combine(comb, w_pad, b.reshape(1, -1))
    return out.reshape(bsz, seq, W.shape[1])
